# baseline (speedup 1.0000x reference)
import jax
import jax.numpy as jnp
from jax.experimental import pallas as pl
from jax.experimental.pallas import tpu as pltpu

N = 5000
K = 12544
H = 1024
BM = 200
NB = N // BM
BKC = 1792
NKC = K // BKC
NSTG = 6
OUT_W = 128


def _boxhead_kernel(x_hbm, w1_hbm, b1_ref, w2_ref, b2_ref, wh_ref, bh_ref,
                    out_hbm, xb, xstage, w1stage, w1b, outbuf,
                    xsem, wsem, osem):
    i = pl.program_id(0)

    def x_copy(r, s, buf):
        return pltpu.make_async_copy(
            x_hbm.at[pl.ds(r * BM, BM), pl.ds(s * BKC, BKC)],
            xstage.at[buf], xsem.at[buf])

    def w1_copy(j):
        return pltpu.make_async_copy(
            w1_hbm.at[pl.ds(j * BKC, BKC), :], w1stage, wsem)

    @pl.when(i == 0)
    def _kickoff():
        for s in range(NSTG):
            x_copy(0, s, s).start()
        w1_copy(0).start()

    @pl.when(i > 0)
    def _compute():
        src = (i + 1) % 2
        h1 = jnp.dot(xb[src], w1b[...], preferred_element_type=jnp.float32)
        h1 = jnp.maximum(h1 + b1_ref[...], 0.0)
        h2 = jnp.dot(h1.astype(jnp.bfloat16), w2_ref[...],
                     preferred_element_type=jnp.float32)
        h2 = jnp.maximum(h2 + b2_ref[...], 0.0)
        out = jnp.dot(h2.astype(jnp.bfloat16), wh_ref[...],
                      preferred_element_type=jnp.float32)
        outbuf[...] = out + bh_ref[...]
        ocp = pltpu.make_async_copy(
            outbuf, out_hbm.at[pl.ds((i - 1) * BM, BM), :], osem)
        ocp.start()
        ocp.wait()

    @pl.when(i < NB)
    def _fill():
        for j in range(NKC):
            buf = (i * NKC + j) % NSTG
            x_copy(i, j, buf).wait()
            xb[i % 2, :, pl.ds(j * BKC, BKC)] = \
                xstage[buf].astype(jnp.bfloat16)
            nxt = j + NSTG
            if nxt < NKC:
                x_copy(i, nxt, buf).start()
            else:
                @pl.when(i < NB - 1)
                def _pf():
                    x_copy(i + 1, nxt - NKC, buf).start()

            @pl.when(i == 0)
            def _convert_w1():
                w1_copy(j).wait()
                cvt = w1stage[...].astype(jnp.bfloat16)
                if j < NKC - 1:
                    w1_copy(j + 1).start()
                w1b[pl.ds(j * BKC, BKC), :] = cvt


def kernel(feature_vectors, W1, b1, W2, b2, Wc, bc, Wr, br):
    n_heads = Wc.shape[1] + Wr.shape[1]
    wh = jnp.concatenate(
        [Wc, Wr, jnp.zeros((H, OUT_W - n_heads), dtype=Wc.dtype)], axis=1)
    bh = jnp.concatenate(
        [bc, br, jnp.zeros((OUT_W - n_heads,), dtype=bc.dtype)])

    w2b = W2.astype(jnp.bfloat16)
    whb = wh.astype(jnp.bfloat16)

    grid = (NB + 1,)
    out = pl.pallas_call(
        _boxhead_kernel,
        grid=grid,
        in_specs=[
            pl.BlockSpec(memory_space=pl.ANY),
            pl.BlockSpec(memory_space=pl.ANY),
            pl.BlockSpec((1, H), lambda i: (0, 0)),
            pl.BlockSpec((H, H), lambda i: (0, 0)),
            pl.BlockSpec((1, H), lambda i: (0, 0)),
            pl.BlockSpec((H, OUT_W), lambda i: (0, 0)),
            pl.BlockSpec((1, OUT_W), lambda i: (0, 0)),
        ],
        out_specs=pl.BlockSpec(memory_space=pl.ANY),
        out_shape=jax.ShapeDtypeStruct((N, OUT_W), jnp.float32),
        scratch_shapes=[
            pltpu.VMEM((2, BM, K), jnp.bfloat16),
            pltpu.VMEM((NSTG, BM, BKC), jnp.float32),
            pltpu.VMEM((BKC, H), jnp.float32),
            pltpu.VMEM((K, H), jnp.bfloat16),
            pltpu.VMEM((BM, OUT_W), jnp.float32),
            pltpu.SemaphoreType.DMA((NSTG,)),
            pltpu.SemaphoreType.DMA,
            pltpu.SemaphoreType.DMA,
        ],
        compiler_params=pltpu.CompilerParams(
            dimension_semantics=("arbitrary",),
            vmem_limit_bytes=67108864,
        ),
    )(feature_vectors, W1, b1.reshape(1, H), w2b, b2.reshape(1, H),
      whb, bh.reshape(1, OUT_W))

    return out[:, :Wc.shape[1]], out[:, Wc.shape[1]:n_heads]


# speedup vs baseline: 1.4129x; 1.4129x over previous
import jax
import jax.numpy as jnp
from jax.experimental import pallas as pl
from jax.experimental.pallas import tpu as pltpu

N = 5000
K = 12544
H = 1024
BM = 1000
BKC = 1792
NKC = K // BKC
NM = N // BM
TAIL_CHUNK = 200
OUT_W = 128


def _boxhead_kernel(x_hbm, w1_hbm, b1_ref, w2_ref, b2_ref, wh_ref, bh_ref,
                    out_ref, xbuf, w1stage, w1b, acc_ref, xsem, wsem):
    m = pl.program_id(0)

    half = BKC // 2

    def x_copy_pair(r, s, buf):
        return (
            pltpu.make_async_copy(
                x_hbm.at[pl.ds(r * BM, BM), pl.ds(s * BKC, half)],
                xbuf.at[buf, :, pl.ds(0, half)], xsem.at[buf, 0]),
            pltpu.make_async_copy(
                x_hbm.at[pl.ds(r * BM, BM), pl.ds(s * BKC + half, half)],
                xbuf.at[buf, :, pl.ds(half, half)], xsem.at[buf, 1]),
        )

    def x_start(r, s, buf):
        for c in x_copy_pair(r, s, buf):
            c.start()

    def x_wait(r, s, buf):
        for c in x_copy_pair(r, s, buf):
            c.wait()

    def w1_copy(j):
        return pltpu.make_async_copy(
            w1_hbm.at[pl.ds(j * BKC, BKC), :], w1stage, wsem)

    @pl.when(m == 0)
    def _kickoff():
        x_start(0, 0, 0)
        x_start(0, 1, 1)
        w1_copy(0).start()

    for j in range(NKC):
        buf = (m * NKC + j) % 3
        nbuf = (m * NKC + j + 1) % 3

        nxt = j + 2
        nnbuf = (m * NKC + nxt) % 3
        if nxt < NKC:
            x_start(m, nxt, nnbuf)
        else:
            @pl.when(m < NM - 1)
            def _pf():
                x_start(m + 1, nxt - NKC, nnbuf)

        @pl.when(m == 0)
        def _convert():
            w1_copy(j).wait()
            w1b[pl.ds(j * BKC, BKC), :] = w1stage[...].astype(jnp.bfloat16)
            if j < NKC - 1:
                w1_copy(j + 1).start()

        x_wait(m, j, buf)
        part = jnp.dot(xbuf[buf].astype(jnp.bfloat16),
                       w1b[pl.ds(j * BKC, BKC), :],
                       preferred_element_type=jnp.float32)
        if j == 0:
            acc_ref[...] = part
        else:
            acc_ref[...] += part

    for t in range(BM // TAIL_CHUNK):
        rows = pl.ds(t * TAIL_CHUNK, TAIL_CHUNK)
        h1 = jnp.maximum(acc_ref[rows, :] + b1_ref[...], 0.0)
        h2 = jnp.dot(h1.astype(jnp.bfloat16), w2_ref[...],
                     preferred_element_type=jnp.float32)
        h2 = jnp.maximum(h2 + b2_ref[...], 0.0)
        out = jnp.dot(h2.astype(jnp.bfloat16), wh_ref[...],
                      preferred_element_type=jnp.float32)
        out_ref[rows, :] = out + bh_ref[...]


def kernel(feature_vectors, W1, b1, W2, b2, Wc, bc, Wr, br):
    n_heads = Wc.shape[1] + Wr.shape[1]
    wh = jnp.concatenate(
        [Wc, Wr, jnp.zeros((H, OUT_W - n_heads), dtype=Wc.dtype)], axis=1)
    bh = jnp.concatenate(
        [bc, br, jnp.zeros((OUT_W - n_heads,), dtype=bc.dtype)])

    w2b = W2.astype(jnp.bfloat16)
    whb = wh.astype(jnp.bfloat16)

    grid = (NM,)
    out = pl.pallas_call(
        _boxhead_kernel,
        grid=grid,
        in_specs=[
            pl.BlockSpec(memory_space=pl.ANY),
            pl.BlockSpec(memory_space=pl.ANY),
            pl.BlockSpec((1, H), lambda m: (0, 0)),
            pl.BlockSpec((H, H), lambda m: (0, 0)),
            pl.BlockSpec((1, H), lambda m: (0, 0)),
            pl.BlockSpec((H, OUT_W), lambda m: (0, 0)),
            pl.BlockSpec((1, OUT_W), lambda m: (0, 0)),
        ],
        out_specs=pl.BlockSpec((BM, OUT_W), lambda m: (m, 0)),
        out_shape=jax.ShapeDtypeStruct((N, OUT_W), jnp.float32),
        scratch_shapes=[
            pltpu.VMEM((3, BM, BKC), jnp.float32),
            pltpu.VMEM((BKC, H), jnp.float32),
            pltpu.VMEM((K, H), jnp.bfloat16),
            pltpu.VMEM((BM, H), jnp.float32),
            pltpu.SemaphoreType.DMA((3, 2)),
            pltpu.SemaphoreType.DMA,
        ],
        compiler_params=pltpu.CompilerParams(
            dimension_semantics=("arbitrary",),
            vmem_limit_bytes=67108864,
        ),
    )(feature_vectors, W1, b1.reshape(1, H), w2b, b2.reshape(1, H),
      whb, bh.reshape(1, OUT_W))

    return out[:, :Wc.shape[1]], out[:, Wc.shape[1]:n_heads]


# speedup vs baseline: 1.4368x; 1.0169x over previous
import jax
import jax.numpy as jnp
from jax.experimental import pallas as pl
from jax.experimental.pallas import tpu as pltpu

N = 5000
K = 12544
H = 1024
BM = 1000
BKC = 1792
NKC = K // BKC
NM = N // BM
TAIL_CHUNK = 200
OUT_W = 128


def _boxhead_kernel(x_hbm, w1_hbm, b1_ref, w2_ref, b2_ref, wh_ref, bh_ref,
                    out_ref, xbuf, w1stage, w1b, acc_ref, xsem, wsem):
    m = pl.program_id(0)

    half = BKC // 2

    def x_copy_pair(r, s, buf):
        return (
            pltpu.make_async_copy(
                x_hbm.at[pl.ds(r * BM, BM), pl.ds(s * BKC, half)],
                xbuf.at[buf, :, pl.ds(0, half)], xsem.at[buf, 0]),
            pltpu.make_async_copy(
                x_hbm.at[pl.ds(r * BM, BM), pl.ds(s * BKC + half, half)],
                xbuf.at[buf, :, pl.ds(half, half)], xsem.at[buf, 1]),
        )

    def x_start(r, s, buf):
        for c in x_copy_pair(r, s, buf):
            c.start()

    def x_wait(r, s, buf):
        for c in x_copy_pair(r, s, buf):
            c.wait()

    def w1_copy(j):
        return pltpu.make_async_copy(
            w1_hbm.at[pl.ds(j * BKC, BKC), :], w1stage, wsem)

    @pl.when(m == 0)
    def _kickoff():
        x_start(0, 0, 0)
        w1_copy(0).start()

    for j in range(NKC):
        buf = (m * NKC + j) % 2
        nbuf = 1 - buf

        if j < NKC - 1:
            x_start(m, j + 1, nbuf)
        else:
            @pl.when(m < NM - 1)
            def _pf():
                x_start(m + 1, 0, nbuf)

        @pl.when(m == 0)
        def _convert():
            w1_copy(j).wait()
            w1b[pl.ds(j * BKC, BKC), :] = w1stage[...].astype(jnp.bfloat16)
            if j < NKC - 1:
                w1_copy(j + 1).start()

        x_wait(m, j, buf)
        part = jnp.dot(xbuf[buf].astype(jnp.bfloat16),
                       w1b[pl.ds(j * BKC, BKC), :],
                       preferred_element_type=jnp.float32)
        if j == 0:
            acc_ref[...] = part
        else:
            acc_ref[...] += part

    for t in range(BM // TAIL_CHUNK):
        rows = pl.ds(t * TAIL_CHUNK, TAIL_CHUNK)
        h1 = jnp.maximum(acc_ref[rows, :] + b1_ref[...], 0.0)
        h2 = jnp.dot(h1.astype(jnp.bfloat16), w2_ref[...],
                     preferred_element_type=jnp.float32)
        h2 = jnp.maximum(h2 + b2_ref[...], 0.0)
        out = jnp.dot(h2.astype(jnp.bfloat16), wh_ref[...],
                      preferred_element_type=jnp.float32)
        out_ref[rows, :] = out + bh_ref[...]


def kernel(feature_vectors, W1, b1, W2, b2, Wc, bc, Wr, br):
    n_heads = Wc.shape[1] + Wr.shape[1]
    wh = jnp.concatenate(
        [Wc, Wr, jnp.zeros((H, OUT_W - n_heads), dtype=Wc.dtype)], axis=1)
    bh = jnp.concatenate(
        [bc, br, jnp.zeros((OUT_W - n_heads,), dtype=bc.dtype)])

    w2b = W2.astype(jnp.bfloat16)
    whb = wh.astype(jnp.bfloat16)

    grid = (NM,)
    out = pl.pallas_call(
        _boxhead_kernel,
        grid=grid,
        in_specs=[
            pl.BlockSpec(memory_space=pl.ANY),
            pl.BlockSpec(memory_space=pl.ANY),
            pl.BlockSpec((1, H), lambda m: (0, 0)),
            pl.BlockSpec((H, H), lambda m: (0, 0)),
            pl.BlockSpec((1, H), lambda m: (0, 0)),
            pl.BlockSpec((H, OUT_W), lambda m: (0, 0)),
            pl.BlockSpec((1, OUT_W), lambda m: (0, 0)),
        ],
        out_specs=pl.BlockSpec((BM, OUT_W), lambda m: (m, 0)),
        out_shape=jax.ShapeDtypeStruct((N, OUT_W), jnp.float32),
        scratch_shapes=[
            pltpu.VMEM((2, BM, BKC), jnp.float32),
            pltpu.VMEM((BKC, H), jnp.float32),
            pltpu.VMEM((K, H), jnp.bfloat16),
            pltpu.VMEM((BM, H), jnp.float32),
            pltpu.SemaphoreType.DMA((2, 2)),
            pltpu.SemaphoreType.DMA,
        ],
        compiler_params=pltpu.CompilerParams(
            dimension_semantics=("arbitrary",),
            vmem_limit_bytes=67108864,
        ),
    )(feature_vectors, W1, b1.reshape(1, H), w2b, b2.reshape(1, H),
      whb, bh.reshape(1, OUT_W))

    return out[:, :Wc.shape[1]], out[:, Wc.shape[1]:n_heads]
